# Initial kernel scaffold; baseline (speedup 1.0000x reference)
import functools

import jax
import jax.numpy as jnp
from jax import lax
from jax.experimental import pallas as pl
from jax.experimental.pallas import tpu as pltpu
from jax.experimental.pallas import tpu_sc as plsc

_INFO = plsc.get_sparse_core_info()
_NC, _NS, _L = _INFO.num_cores, _INFO.num_subcores, _INFO.num_lanes
_NW = _NC * _NS

_BATCH = 16384
_COLS = 128
_ROWS = _BATCH // _COLS
_ROWS_PER_W = _ROWS // _NW
_WEEK_PAD = 1024


def _fe_kernel(vidx_hbm, widx_hbm, lc_hbm, vtab_hbm, wtab_hbm, beta_hbm,
               out_hbm, vidx_v, widx_v, lc_v, veff_v, wtab_v, beta_v, out_v,
               sem):
    wid = lax.axis_index("s") * _NC + lax.axis_index("c")
    base = wid * _ROWS_PER_W

    pltpu.sync_copy(vidx_hbm.at[pl.ds(base, _ROWS_PER_W)], vidx_v)
    gathers = [
        pltpu.async_copy(vtab_hbm.at[vidx_v.at[j]], veff_v.at[j], sem)
        for j in range(_ROWS_PER_W)
    ]

    pltpu.sync_copy(widx_hbm.at[pl.ds(base, _ROWS_PER_W)], widx_v)
    pltpu.sync_copy(lc_hbm.at[pl.ds(base, _ROWS_PER_W)], lc_v)
    pltpu.sync_copy(wtab_hbm, wtab_v)
    pltpu.sync_copy(beta_hbm, beta_v)
    for g in gathers:
        g.wait()

    b16 = beta_v[...]
    for r in range(_ROWS_PER_W):
        for c in range(_COLS // _L):
            sl = pl.ds(c * _L, _L)
            w16 = plsc.load_gather(wtab_v, [widx_v[r, sl]])
            out_v[r, sl] = lc_v[r, sl] * b16 + veff_v[r, sl] + w16

    pltpu.sync_copy(out_v, out_hbm.at[pl.ds(base, _ROWS_PER_W)])


@jax.jit
def _run(vidx2, widx2, lc2, vtab, wtab, beta16):
    mesh = plsc.VectorSubcoreMesh(core_axis_name="c", subcore_axis_name="s")
    f = functools.partial(
        pl.kernel,
        out_type=jax.ShapeDtypeStruct((_ROWS, _COLS), jnp.float32),
        mesh=mesh,
        scratch_types=[
            pltpu.VMEM((_ROWS_PER_W, _COLS), jnp.int32),
            pltpu.VMEM((_ROWS_PER_W, _COLS), jnp.int32),
            pltpu.VMEM((_ROWS_PER_W, _COLS), jnp.float32),
            pltpu.VMEM((_ROWS_PER_W, _COLS), jnp.float32),
            pltpu.VMEM((_WEEK_PAD,), jnp.float32),
            pltpu.VMEM((_L,), jnp.float32),
            pltpu.VMEM((_ROWS_PER_W, _COLS), jnp.float32),
            pltpu.SemaphoreType.DMA,
        ],
    )(_fe_kernel)
    return f(vidx2, widx2, lc2, vtab, wtab, beta16)


def kernel(vendor_ids, week_ids, log_clicks, vendor_fe, week_fe, beta):
    vidx2 = vendor_ids.astype(jnp.int32).reshape(_ROWS, _COLS)
    widx2 = week_ids.astype(jnp.int32).reshape(_ROWS, _COLS)
    lc2 = log_clicks.reshape(_ROWS, _COLS)
    vtab = vendor_fe.reshape(-1)
    wtab = jnp.zeros((_WEEK_PAD,), jnp.float32).at[: week_fe.shape[0]].set(
        week_fe.reshape(-1))
    beta16 = jnp.broadcast_to(beta.astype(jnp.float32), (_L,))
    out2 = _run(vidx2, widx2, lc2, vtab, wtab, beta16)
    return out2.reshape(_BATCH)


# speedup vs baseline: 2.1310x; 2.1310x over previous
import functools

import jax
import jax.numpy as jnp
from jax import lax
from jax.experimental import pallas as pl
from jax.experimental.pallas import tpu as pltpu
from jax.experimental.pallas import tpu_sc as plsc

_INFO = plsc.get_sparse_core_info()
_NC, _NS, _L = _INFO.num_cores, _INFO.num_subcores, _INFO.num_lanes
_NW = _NC * _NS

_BATCH = 16384
_COLS = 128
_ROWS = _BATCH // _COLS
_ROWS_PER_W = _ROWS // _NW
_WEEK_PAD = 1024


def _fe_kernel(vidx_hbm, widx_hbm, lc_hbm, vtab_hbm, wtab_hbm, beta_hbm,
               out_hbm, vidx_v, widx_v, lc_v, veff_v, weff_v, beta_v, out_v,
               sem):
    wid = lax.axis_index("s") * _NC + lax.axis_index("c")
    base = wid * _ROWS_PER_W

    pltpu.sync_copy(vidx_hbm.at[pl.ds(base, _ROWS_PER_W)], vidx_v)
    pltpu.sync_copy(widx_hbm.at[pl.ds(base, _ROWS_PER_W)], widx_v)
    gathers = [
        pltpu.async_copy(vtab_hbm.at[vidx_v.at[j]], veff_v.at[j], sem)
        for j in range(_ROWS_PER_W)
    ] + [
        pltpu.async_copy(wtab_hbm.at[widx_v.at[j]], weff_v.at[j], sem)
        for j in range(_ROWS_PER_W)
    ]

    pltpu.sync_copy(lc_hbm.at[pl.ds(base, _ROWS_PER_W)], lc_v)
    pltpu.sync_copy(beta_hbm, beta_v)
    for g in gathers:
        g.wait()

    b16 = beta_v[...]
    for r in range(_ROWS_PER_W):
        for c in range(_COLS // _L):
            sl = pl.ds(c * _L, _L)
            out_v[r, sl] = lc_v[r, sl] * b16 + veff_v[r, sl] + weff_v[r, sl]

    pltpu.sync_copy(out_v, out_hbm.at[pl.ds(base, _ROWS_PER_W)])


@jax.jit
def _run(vidx2, widx2, lc2, vtab, wtab, beta16):
    mesh = plsc.VectorSubcoreMesh(core_axis_name="c", subcore_axis_name="s")
    f = functools.partial(
        pl.kernel,
        out_type=jax.ShapeDtypeStruct((_ROWS, _COLS), jnp.float32),
        mesh=mesh,
        scratch_types=[
            pltpu.VMEM((_ROWS_PER_W, _COLS), jnp.int32),
            pltpu.VMEM((_ROWS_PER_W, _COLS), jnp.int32),
            pltpu.VMEM((_ROWS_PER_W, _COLS), jnp.float32),
            pltpu.VMEM((_ROWS_PER_W, _COLS), jnp.float32),
            pltpu.VMEM((_ROWS_PER_W, _COLS), jnp.float32),
            pltpu.VMEM((_L,), jnp.float32),
            pltpu.VMEM((_ROWS_PER_W, _COLS), jnp.float32),
            pltpu.SemaphoreType.DMA,
        ],
    )(_fe_kernel)
    return f(vidx2, widx2, lc2, vtab, wtab, beta16)


def kernel(vendor_ids, week_ids, log_clicks, vendor_fe, week_fe, beta):
    vidx2 = vendor_ids.astype(jnp.int32).reshape(_ROWS, _COLS)
    widx2 = week_ids.astype(jnp.int32).reshape(_ROWS, _COLS)
    lc2 = log_clicks.reshape(_ROWS, _COLS)
    vtab = vendor_fe.reshape(-1)
    wtab = jnp.zeros((_WEEK_PAD,), jnp.float32).at[: week_fe.shape[0]].set(
        week_fe.reshape(-1))
    beta16 = jnp.broadcast_to(beta.astype(jnp.float32), (_L,))
    out2 = _run(vidx2, widx2, lc2, vtab, wtab, beta16)
    return out2.reshape(_BATCH)


# speedup vs baseline: 4.1076x; 1.9276x over previous
import functools

import jax
import jax.numpy as jnp
from jax import lax
from jax.experimental import pallas as pl
from jax.experimental.pallas import tpu as pltpu
from jax.experimental.pallas import tpu_sc as plsc

_INFO = plsc.get_sparse_core_info()
_NC, _NS, _L = _INFO.num_cores, _INFO.num_subcores, _INFO.num_lanes
_NW = _NC * _NS

_BATCH = 16384
_COLS = 128
_ROWS = _BATCH // _COLS
_ROWS_PER_W = _ROWS // _NW
_N_VENDORS = 1000000
_VTAB_PAD = 1000448


def _fe_kernel(vidx_hbm, widx_hbm, lc_hbm, vtab_hbm, wtab_hbm, beta_hbm,
               out_hbm, vidx_v, widx_v, lc_v, veff_v, weff_v, beta_v, out_v,
               sem):
    wid = lax.axis_index("s") * _NC + lax.axis_index("c")
    base = wid * _ROWS_PER_W

    pltpu.sync_copy(vidx_hbm.at[pl.ds(base, _ROWS_PER_W)], vidx_v)
    pltpu.sync_copy(widx_hbm.at[pl.ds(base, _ROWS_PER_W)], widx_v)
    gathers = [
        pltpu.async_copy(vtab_hbm.at[vidx_v.at[j]], veff_v.at[j], sem)
        for j in range(_ROWS_PER_W)
    ] + [
        pltpu.async_copy(wtab_hbm.at[widx_v.at[j]], weff_v.at[j], sem)
        for j in range(_ROWS_PER_W)
    ]

    pltpu.sync_copy(lc_hbm.at[pl.ds(base, _ROWS_PER_W)], lc_v)
    pltpu.sync_copy(beta_hbm, beta_v)
    for g in gathers:
        g.wait()

    b16 = beta_v[...]
    for r in range(_ROWS_PER_W):
        for c in range(_COLS // _L):
            sl = pl.ds(c * _L, _L)
            out_v[r, sl] = lc_v[r, sl] * b16 + veff_v[r, sl] + weff_v[r, sl]

    pltpu.sync_copy(out_v, out_hbm.at[pl.ds(base, _ROWS_PER_W)])


@jax.jit
def _run(vidx2, widx2, lc2, vtab, wtab, beta16):
    mesh = plsc.VectorSubcoreMesh(core_axis_name="c", subcore_axis_name="s")
    f = functools.partial(
        pl.kernel,
        out_type=jax.ShapeDtypeStruct((_ROWS, _COLS), jnp.float32),
        mesh=mesh,
        scratch_types=[
            pltpu.VMEM((_ROWS_PER_W, _COLS), jnp.int32),
            pltpu.VMEM((_ROWS_PER_W, _COLS), jnp.int32),
            pltpu.VMEM((_ROWS_PER_W, _COLS), jnp.float32),
            pltpu.VMEM((_ROWS_PER_W, _COLS), jnp.float32),
            pltpu.VMEM((_ROWS_PER_W, _COLS), jnp.float32),
            pltpu.VMEM((_L,), jnp.float32),
            pltpu.VMEM((_ROWS_PER_W, _COLS), jnp.float32),
            pltpu.SemaphoreType.DMA,
        ],
    )(_fe_kernel)
    return f(vidx2, widx2, lc2, vtab, wtab, beta16)


def kernel(vendor_ids, week_ids, log_clicks, vendor_fe, week_fe, beta):
    vidx2 = vendor_ids.astype(jnp.int32).reshape(_ROWS, _COLS)
    widx2 = week_ids.astype(jnp.int32).reshape(_ROWS, _COLS)
    lc2 = log_clicks.reshape(_ROWS, _COLS)
    vtab = jnp.pad(vendor_fe, ((0, _VTAB_PAD - _N_VENDORS), (0, 0))).reshape(-1)
    wtab = week_fe.reshape(-1)
    beta16 = jnp.broadcast_to(beta.astype(jnp.float32), (_L,))
    out2 = _run(vidx2, widx2, lc2, vtab, wtab, beta16)
    return out2.reshape(_BATCH)
